# baseline (speedup 1.0000x reference)
import functools

import jax
import jax.numpy as jnp
from jax import lax
import jax.experimental.pallas as pl
from jax.experimental.pallas import tpu as pltpu
from jax.experimental.pallas import tpu_sc as plsc

_LANES = 16
_LO_BITS = 12
_HI_BITS = 11
_NLO = 1 << _LO_BITS
_NHI = 1 << _HI_BITS
_SCALE = float(1 << (_LO_BITS + _HI_BITS))
_IDX_BITS = 15


def _row_body(nbuf, buf1, rankb, keepb, hla, hlb, hha, hhb, t, n_keep):
    t2 = t // 2
    nv2 = t2 // _LANES
    lanes = lax.iota(jnp.int32, _LANES)
    zeros_i = jnp.zeros((_LANES,), jnp.int32)
    ones_f = jnp.full((_LANES,), 1.0, jnp.float32)
    zeros_f = jnp.zeros((_LANES,), jnp.float32)

    def zero_lo(i, c):
        hla[pl.ds(i * _LANES, _LANES)] = zeros_i
        hlb[pl.ds(i * _LANES, _LANES)] = zeros_i
        return c

    def zero_hi(i, c):
        hha[pl.ds(i * _LANES, _LANES)] = zeros_i
        hhb[pl.ds(i * _LANES, _LANES)] = zeros_i
        return c

    lax.fori_loop(0, _NLO // _LANES, zero_lo, 0, unroll=4)
    lax.fori_loop(0, _NHI // _LANES, zero_hi, 0, unroll=4)

    def hist_body(i, c):
        va = nbuf[pl.ds(i * _LANES, _LANES)]
        vb = nbuf[pl.ds(t2 + i * _LANES, _LANES)]
        loa = jnp.bitwise_and((va * _SCALE).astype(jnp.int32), _NLO - 1)
        lob = jnp.bitwise_and((vb * _SCALE).astype(jnp.int32), _NLO - 1)
        ca, ma = plsc.scan_count(loa)
        plsc.addupdate_scatter(hla, [loa], ca, mask=ma)
        cb, mb = plsc.scan_count(lob)
        plsc.addupdate_scatter(hlb, [lob], cb, mask=mb)
        return c

    lax.fori_loop(0, nv2, hist_body, 0, unroll=2)

    def scan_body(ha, hb, i, carry):
        xa = ha[pl.ds(i * _LANES, _LANES)]
        xb = hb[pl.ds(i * _LANES, _LANES)]
        s = xa + xb
        inc = plsc.cumsum(s)
        start = inc - s + carry
        ha[pl.ds(i * _LANES, _LANES)] = start
        hb[pl.ds(i * _LANES, _LANES)] = start + xa
        return carry + jnp.sum(s)

    lax.fori_loop(0, _NLO // _LANES, functools.partial(scan_body, hla, hlb),
                  jnp.int32(0))

    def pass1_half(v, lo, hi, idxv, hl):
        cnt, ml = plsc.scan_count(lo)
        base = plsc.load_gather(hl, [lo])
        pos = base + cnt - 1
        packed = jnp.bitwise_or(jnp.left_shift(hi, _IDX_BITS), idxv)
        plsc.store_scatter(buf1, [pos], packed)
        plsc.store_scatter(hl, [lo], pos + 1, mask=ml)
        in_a = pos < t2
        c1, m1 = plsc.scan_count(hi, in_a)
        plsc.addupdate_scatter(hha, [hi], c1, mask=m1)
        c2, m2 = plsc.scan_count(hi, jnp.logical_not(in_a))
        plsc.addupdate_scatter(hhb, [hi], c2, mask=m2)

    def pass1_body(i, c):
        va = nbuf[pl.ds(i * _LANES, _LANES)]
        vb = nbuf[pl.ds(t2 + i * _LANES, _LANES)]
        ika = (va * _SCALE).astype(jnp.int32)
        ikb = (vb * _SCALE).astype(jnp.int32)
        pass1_half(va, jnp.bitwise_and(ika, _NLO - 1),
                   jnp.right_shift(ika, _LO_BITS), lanes + i * _LANES, hla)
        pass1_half(vb, jnp.bitwise_and(ikb, _NLO - 1),
                   jnp.right_shift(ikb, _LO_BITS), lanes + t2 + i * _LANES,
                   hlb)
        return c

    lax.fori_loop(0, nv2, pass1_body, 0, unroll=2)

    lax.fori_loop(0, _NHI // _LANES, functools.partial(scan_body, hha, hhb),
                  jnp.int32(0))

    def pass2_half(p, hh):
        hi = jnp.right_shift(p, _IDX_BITS)
        idxv = jnp.bitwise_and(p, (1 << _IDX_BITS) - 1)
        cnt, mh = plsc.scan_count(hi)
        base = plsc.load_gather(hh, [hi])
        rank = base + cnt - 1
        plsc.store_scatter(rankb, [idxv], rank)
        plsc.store_scatter(keepb, [rank], idxv, mask=rank < n_keep)
        plsc.store_scatter(nbuf, [idxv],
                           jnp.where(rank >= n_keep, ones_f, zeros_f))
        plsc.store_scatter(hh, [hi], rank + 1, mask=mh)

    def pass2_body(i, c):
        pass2_half(buf1[pl.ds(i * _LANES, _LANES)], hha)
        pass2_half(buf1[pl.ds(t2 + i * _LANES, _LANES)], hhb)
        return c

    lax.fori_loop(0, nv2, pass2_body, 0, unroll=2)


def _make_sc_kernel(b, t):
    n_keep = t // 2
    rows_per_w = b // 32
    mesh = plsc.VectorSubcoreMesh(core_axis_name="c", subcore_axis_name="s")

    @functools.partial(
        pl.kernel,
        out_type=(
            jax.ShapeDtypeStruct((b * n_keep,), jnp.int32),
            jax.ShapeDtypeStruct((b * t,), jnp.int32),
            jax.ShapeDtypeStruct((b * t,), jnp.float32),
        ),
        mesh=mesh,
        scratch_types=[
            pltpu.VMEM((t,), jnp.float32),
            pltpu.VMEM((t,), jnp.int32),
            pltpu.VMEM((t,), jnp.int32),
            pltpu.VMEM((n_keep,), jnp.int32),
            pltpu.VMEM((_NLO,), jnp.int32),
            pltpu.VMEM((_NLO,), jnp.int32),
            pltpu.VMEM((_NHI,), jnp.int32),
            pltpu.VMEM((_NHI,), jnp.int32),
        ],
        compiler_params=pltpu.CompilerParams(needs_layout_passes=False),
    )
    def sc_kernel(noise_hbm, keep_o, restore_o, mask_o,
                  nbuf, buf1, rankb, keepb, hla, hlb, hha, hhb):
        wid = lax.axis_index("s") * 2 + lax.axis_index("c")

        def do_row(r, c):
            row = wid * rows_per_w + r
            pltpu.sync_copy(noise_hbm.at[pl.ds(row * t, t)], nbuf)
            _row_body(nbuf, buf1, rankb, keepb, hla, hlb, hha, hhb, t,
                      n_keep)
            pltpu.sync_copy(rankb, restore_o.at[pl.ds(row * t, t)])
            pltpu.sync_copy(keepb, keep_o.at[pl.ds(row * n_keep, n_keep)])
            pltpu.sync_copy(nbuf, mask_o.at[pl.ds(row * t, t)])
            return c

        lax.fori_loop(0, rows_per_w, do_row, 0)

    return sc_kernel


def kernel(B, T, noise):
    b, t = noise.shape
    n_keep = t // 2
    keep, restore, mask = _make_sc_kernel(b, t)(noise.reshape(-1))
    return (keep.reshape(b, n_keep), restore.reshape(b, t),
            mask.reshape(b, t))


# speedup vs baseline: 1.1319x; 1.1319x over previous
import functools

import jax
import jax.numpy as jnp
from jax import lax
import jax.experimental.pallas as pl
from jax.experimental.pallas import tpu as pltpu
from jax.experimental.pallas import tpu_sc as plsc

_LANES = 16
_LO_BITS = 12
_HI_BITS = 11
_NLO = 1 << _LO_BITS
_NHI = 1 << _HI_BITS
_SCALE = float(1 << (_LO_BITS + _HI_BITS))
_IDX_BITS = 15


def _row_body(nbuf, buf1, rankb, keepb, hla, hlb, hha, hhb, t, n_keep):
    t2 = t // 2
    nv2 = t2 // _LANES
    lanes = lax.iota(jnp.int32, _LANES)
    zeros_i = jnp.zeros((_LANES,), jnp.int32)
    ones_f = jnp.full((_LANES,), 1.0, jnp.float32)
    zeros_f = jnp.zeros((_LANES,), jnp.float32)

    @plsc.parallel_loop(0, _NLO // _LANES, unroll=8)
    def zero_lo(i):
        hla[pl.ds(i * _LANES, _LANES)] = zeros_i
        hlb[pl.ds(i * _LANES, _LANES)] = zeros_i

    @plsc.parallel_loop(0, _NHI // _LANES, unroll=8)
    def zero_hi(i):
        hha[pl.ds(i * _LANES, _LANES)] = zeros_i
        hhb[pl.ds(i * _LANES, _LANES)] = zeros_i

    @plsc.parallel_loop(0, nv2, unroll=4)
    def hist_body(i):
        va = nbuf[pl.ds(i * _LANES, _LANES)]
        vb = nbuf[pl.ds(t2 + i * _LANES, _LANES)]
        loa = jnp.bitwise_and((va * _SCALE).astype(jnp.int32), _NLO - 1)
        lob = jnp.bitwise_and((vb * _SCALE).astype(jnp.int32), _NLO - 1)
        ca, ma = plsc.scan_count(loa)
        plsc.addupdate_scatter(hla, [loa], ca, mask=ma)
        cb, mb = plsc.scan_count(lob)
        plsc.addupdate_scatter(hlb, [lob], cb, mask=mb)

    def scan_body(ha, hb, i, carry):
        xa = ha[pl.ds(i * _LANES, _LANES)]
        xb = hb[pl.ds(i * _LANES, _LANES)]
        s = xa + xb
        inc = plsc.cumsum(s)
        start = inc - s + carry
        ha[pl.ds(i * _LANES, _LANES)] = start
        hb[pl.ds(i * _LANES, _LANES)] = start + xa
        return carry + jnp.sum(s)

    lax.fori_loop(0, _NLO // _LANES, functools.partial(scan_body, hla, hlb),
                  jnp.int32(0))

    def pass1_half(v, lo, hi, idxv, hl):
        cnt, ml = plsc.scan_count(lo)
        base = plsc.load_gather(hl, [lo])
        pos = base + cnt - 1
        packed = jnp.bitwise_or(jnp.left_shift(hi, _IDX_BITS), idxv)
        plsc.store_scatter(buf1, [pos], packed)
        plsc.store_scatter(hl, [lo], pos + 1, mask=ml)
        in_a = pos < t2
        c1, m1 = plsc.scan_count(hi, in_a)
        plsc.addupdate_scatter(hha, [hi], c1, mask=m1)
        c2, m2 = plsc.scan_count(hi, jnp.logical_not(in_a))
        plsc.addupdate_scatter(hhb, [hi], c2, mask=m2)

    def pass1_body(i, c):
        va = nbuf[pl.ds(i * _LANES, _LANES)]
        vb = nbuf[pl.ds(t2 + i * _LANES, _LANES)]
        ika = (va * _SCALE).astype(jnp.int32)
        ikb = (vb * _SCALE).astype(jnp.int32)
        pass1_half(va, jnp.bitwise_and(ika, _NLO - 1),
                   jnp.right_shift(ika, _LO_BITS), lanes + i * _LANES, hla)
        pass1_half(vb, jnp.bitwise_and(ikb, _NLO - 1),
                   jnp.right_shift(ikb, _LO_BITS), lanes + t2 + i * _LANES,
                   hlb)
        return c

    lax.fori_loop(0, nv2, pass1_body, 0, unroll=4)

    lax.fori_loop(0, _NHI // _LANES, functools.partial(scan_body, hha, hhb),
                  jnp.int32(0))

    def pass2_half(p, hh):
        hi = jnp.right_shift(p, _IDX_BITS)
        idxv = jnp.bitwise_and(p, (1 << _IDX_BITS) - 1)
        cnt, mh = plsc.scan_count(hi)
        base = plsc.load_gather(hh, [hi])
        rank = base + cnt - 1
        plsc.store_scatter(rankb, [idxv], rank)
        plsc.store_scatter(keepb, [rank], idxv, mask=rank < n_keep)
        plsc.store_scatter(nbuf, [idxv],
                           jnp.where(rank >= n_keep, ones_f, zeros_f))
        plsc.store_scatter(hh, [hi], rank + 1, mask=mh)

    def pass2_body(i, c):
        pass2_half(buf1[pl.ds(i * _LANES, _LANES)], hha)
        pass2_half(buf1[pl.ds(t2 + i * _LANES, _LANES)], hhb)
        return c

    lax.fori_loop(0, nv2, pass2_body, 0, unroll=4)


def _make_sc_kernel(b, t):
    n_keep = t // 2
    rows_per_w = b // 32
    mesh = plsc.VectorSubcoreMesh(core_axis_name="c", subcore_axis_name="s")

    @functools.partial(
        pl.kernel,
        out_type=(
            jax.ShapeDtypeStruct((b * n_keep,), jnp.int32),
            jax.ShapeDtypeStruct((b * t,), jnp.int32),
            jax.ShapeDtypeStruct((b * t,), jnp.float32),
        ),
        mesh=mesh,
        scratch_types=[
            pltpu.VMEM((t,), jnp.float32),
            pltpu.VMEM((t,), jnp.int32),
            pltpu.VMEM((t,), jnp.int32),
            pltpu.VMEM((n_keep,), jnp.int32),
            pltpu.VMEM((_NLO,), jnp.int32),
            pltpu.VMEM((_NLO,), jnp.int32),
            pltpu.VMEM((_NHI,), jnp.int32),
            pltpu.VMEM((_NHI,), jnp.int32),
        ],
        compiler_params=pltpu.CompilerParams(needs_layout_passes=False),
    )
    def sc_kernel(noise_hbm, keep_o, restore_o, mask_o,
                  nbuf, buf1, rankb, keepb, hla, hlb, hha, hhb):
        wid = lax.axis_index("s") * 2 + lax.axis_index("c")

        def do_row(r, c):
            row = wid * rows_per_w + r
            pltpu.sync_copy(noise_hbm.at[pl.ds(row * t, t)], nbuf)
            _row_body(nbuf, buf1, rankb, keepb, hla, hlb, hha, hhb, t,
                      n_keep)
            pltpu.sync_copy(rankb, restore_o.at[pl.ds(row * t, t)])
            pltpu.sync_copy(keepb, keep_o.at[pl.ds(row * n_keep, n_keep)])
            pltpu.sync_copy(nbuf, mask_o.at[pl.ds(row * t, t)])
            return c

        lax.fori_loop(0, rows_per_w, do_row, 0)

    return sc_kernel


def kernel(B, T, noise):
    b, t = noise.shape
    n_keep = t // 2
    keep, restore, mask = _make_sc_kernel(b, t)(noise.reshape(-1))
    return (keep.reshape(b, n_keep), restore.reshape(b, t),
            mask.reshape(b, t))


# speedup vs baseline: 1.3130x; 1.1600x over previous
import functools

import jax
import jax.numpy as jnp
from jax import lax
import jax.experimental.pallas as pl
from jax.experimental.pallas import tpu as pltpu
from jax.experimental.pallas import tpu_sc as plsc

_LANES = 16
_LO_BITS = 12
_HI_BITS = 11
_NLO = 1 << _LO_BITS
_NHI = 1 << _HI_BITS
_SCALE = float(1 << (_LO_BITS + _HI_BITS))
_IDX_BITS = 15


def _row_body(nbuf, buf1, rankb, keepb, hla, hlb, hha, hhb, t, n_keep):
    t2 = t // 2
    nv2 = t2 // _LANES
    lanes = lax.iota(jnp.int32, _LANES)
    zeros_i = jnp.zeros((_LANES,), jnp.int32)
    ones_f = jnp.full((_LANES,), 1.0, jnp.float32)
    zeros_f = jnp.zeros((_LANES,), jnp.float32)

    @plsc.parallel_loop(0, _NLO // _LANES, unroll=8)
    def zero_lo(i):
        hla[pl.ds(i * _LANES, _LANES)] = zeros_i
        hlb[pl.ds(i * _LANES, _LANES)] = zeros_i

    @plsc.parallel_loop(0, _NHI // _LANES, unroll=8)
    def zero_hi(i):
        hha[pl.ds(i * _LANES, _LANES)] = zeros_i
        hhb[pl.ds(i * _LANES, _LANES)] = zeros_i

    @plsc.parallel_loop(0, nv2, unroll=4)
    def hist_body(i):
        va = nbuf[pl.ds(i * _LANES, _LANES)]
        vb = nbuf[pl.ds(t2 + i * _LANES, _LANES)]
        loa = jnp.bitwise_and((va * _SCALE).astype(jnp.int32), _NLO - 1)
        lob = jnp.bitwise_and((vb * _SCALE).astype(jnp.int32), _NLO - 1)
        ca, ma = plsc.scan_count(loa)
        plsc.addupdate_scatter(hla, [loa], ca, mask=ma)
        cb, mb = plsc.scan_count(lob)
        plsc.addupdate_scatter(hlb, [lob], cb, mask=mb)

    def scan_body(ha, hb, i, carry):
        xa = ha[pl.ds(i * _LANES, _LANES)]
        xb = hb[pl.ds(i * _LANES, _LANES)]
        s = xa + xb
        inc = plsc.cumsum(s)
        start = inc - s + carry
        ha[pl.ds(i * _LANES, _LANES)] = start
        hb[pl.ds(i * _LANES, _LANES)] = start + xa
        return carry + jnp.sum(s)

    lax.fori_loop(0, _NLO // _LANES, functools.partial(scan_body, hla, hlb),
                  jnp.int32(0))

    def pass1_half(v, lo, hi, idxv, hl):
        cnt, ml = plsc.scan_count(lo)
        base = plsc.load_gather(hl, [lo])
        pos = base + cnt - 1
        packed = jnp.bitwise_or(jnp.left_shift(hi, _IDX_BITS), idxv)
        plsc.store_scatter(buf1, [pos], packed)
        plsc.store_scatter(hl, [lo], pos + 1, mask=ml)
        in_a = pos < t2
        c1, m1 = plsc.scan_count(hi, in_a)
        plsc.addupdate_scatter(hha, [hi], c1, mask=m1)
        c2, m2 = plsc.scan_count(hi, jnp.logical_not(in_a))
        plsc.addupdate_scatter(hhb, [hi], c2, mask=m2)

    def pass1_body(i, c):
        va = nbuf[pl.ds(i * _LANES, _LANES)]
        vb = nbuf[pl.ds(t2 + i * _LANES, _LANES)]
        ika = (va * _SCALE).astype(jnp.int32)
        ikb = (vb * _SCALE).astype(jnp.int32)
        pass1_half(va, jnp.bitwise_and(ika, _NLO - 1),
                   jnp.right_shift(ika, _LO_BITS), lanes + i * _LANES, hla)
        pass1_half(vb, jnp.bitwise_and(ikb, _NLO - 1),
                   jnp.right_shift(ikb, _LO_BITS), lanes + t2 + i * _LANES,
                   hlb)
        return c

    lax.fori_loop(0, nv2, pass1_body, 0, unroll=8)

    lax.fori_loop(0, _NHI // _LANES, functools.partial(scan_body, hha, hhb),
                  jnp.int32(0))

    def pass2_half(p, hh):
        hi = jnp.right_shift(p, _IDX_BITS)
        idxv = jnp.bitwise_and(p, (1 << _IDX_BITS) - 1)
        cnt, mh = plsc.scan_count(hi)
        base = plsc.load_gather(hh, [hi])
        rank = base + cnt - 1
        plsc.store_scatter(rankb, [idxv], rank)
        plsc.store_scatter(keepb, [rank], idxv, mask=rank < n_keep)
        plsc.store_scatter(nbuf, [idxv],
                           jnp.where(rank >= n_keep, ones_f, zeros_f))
        plsc.store_scatter(hh, [hi], rank + 1, mask=mh)

    def pass2_body(i, c):
        pass2_half(buf1[pl.ds(i * _LANES, _LANES)], hha)
        pass2_half(buf1[pl.ds(t2 + i * _LANES, _LANES)], hhb)
        return c

    lax.fori_loop(0, nv2, pass2_body, 0, unroll=8)


def _make_sc_kernel(b, t):
    n_keep = t // 2
    rows_per_w = b // 32
    mesh = plsc.VectorSubcoreMesh(core_axis_name="c", subcore_axis_name="s")

    @functools.partial(
        pl.kernel,
        out_type=(
            jax.ShapeDtypeStruct((b, n_keep), jnp.int32),
            jax.ShapeDtypeStruct((b, t), jnp.int32),
            jax.ShapeDtypeStruct((b, t), jnp.float32),
        ),
        mesh=mesh,
        scratch_types=[
            pltpu.VMEM((t,), jnp.float32),
            pltpu.VMEM((t,), jnp.int32),
            pltpu.VMEM((t,), jnp.int32),
            pltpu.VMEM((n_keep,), jnp.int32),
            pltpu.VMEM((_NLO,), jnp.int32),
            pltpu.VMEM((_NLO,), jnp.int32),
            pltpu.VMEM((_NHI,), jnp.int32),
            pltpu.VMEM((_NHI,), jnp.int32),
        ],
        compiler_params=pltpu.CompilerParams(needs_layout_passes=False),
    )
    def sc_kernel(noise_hbm, keep_o, restore_o, mask_o,
                  nbuf, buf1, rankb, keepb, hla, hlb, hha, hhb):
        wid = lax.axis_index("s") * 2 + lax.axis_index("c")

        def do_row(r, c):
            row = wid * rows_per_w + r
            pltpu.sync_copy(noise_hbm.at[row], nbuf)
            _row_body(nbuf, buf1, rankb, keepb, hla, hlb, hha, hhb, t,
                      n_keep)
            pltpu.sync_copy(rankb, restore_o.at[row])
            pltpu.sync_copy(keepb, keep_o.at[row])
            pltpu.sync_copy(nbuf, mask_o.at[row])
            return c

        lax.fori_loop(0, rows_per_w, do_row, 0)

    return sc_kernel


def kernel(B, T, noise):
    b, t = noise.shape
    return _make_sc_kernel(b, t)(noise)


# speedup vs baseline: 1.5434x; 1.1755x over previous
import functools

import jax
import jax.numpy as jnp
from jax import lax
import jax.experimental.pallas as pl
from jax.experimental.pallas import tpu as pltpu
from jax.experimental.pallas import tpu_sc as plsc

_LANES = 16
_LO_BITS = 12
_HI_BITS = 11
_NLO = 1 << _LO_BITS
_NHI = 1 << _HI_BITS
_SCALE = float(1 << (_LO_BITS + _HI_BITS))
_IDX_BITS = 15


def _row_body(nbuf, buf1, rankb, keepb, hla, hlb, hha, hhb, t, n_keep):
    t2 = t // 2
    nv2 = t2 // _LANES
    lanes = lax.iota(jnp.int32, _LANES)
    zeros_i = jnp.zeros((_LANES,), jnp.int32)
    ones_f = jnp.full((_LANES,), 1.0, jnp.float32)
    zeros_f = jnp.zeros((_LANES,), jnp.float32)

    @plsc.parallel_loop(0, _NLO // _LANES, unroll=8)
    def zero_lo(i):
        hla[pl.ds(i * _LANES, _LANES)] = zeros_i
        hlb[pl.ds(i * _LANES, _LANES)] = zeros_i

    @plsc.parallel_loop(0, _NHI // _LANES, unroll=8)
    def zero_hi(i):
        hha[pl.ds(i * _LANES, _LANES)] = zeros_i
        hhb[pl.ds(i * _LANES, _LANES)] = zeros_i

    @plsc.parallel_loop(0, nv2, unroll=4)
    def hist_body(i):
        va = nbuf[pl.ds(i * _LANES, _LANES)]
        vb = nbuf[pl.ds(t2 + i * _LANES, _LANES)]
        loa = jnp.bitwise_and((va * _SCALE).astype(jnp.int32), _NLO - 1)
        lob = jnp.bitwise_and((vb * _SCALE).astype(jnp.int32), _NLO - 1)
        ca, ma = plsc.scan_count(loa)
        plsc.addupdate_scatter(hla, [loa], ca, mask=ma)
        cb, mb = plsc.scan_count(lob)
        plsc.addupdate_scatter(hlb, [lob], cb, mask=mb)

    def scan_body(ha, hb, i, carry):
        xa = ha[pl.ds(i * _LANES, _LANES)]
        xb = hb[pl.ds(i * _LANES, _LANES)]
        s = xa + xb
        inc = plsc.cumsum(s)
        start = inc - s + carry
        ha[pl.ds(i * _LANES, _LANES)] = start
        hb[pl.ds(i * _LANES, _LANES)] = start + xa
        return carry + jnp.sum(s)

    lax.fori_loop(0, _NLO // _LANES, functools.partial(scan_body, hla, hlb),
                  jnp.int32(0))

    def pass1_half(v, lo, hi, idxv, hl):
        cnt, ml = plsc.scan_count(lo)
        base = plsc.load_gather(hl, [lo])
        pos = base + cnt - 1
        packed = jnp.bitwise_or(jnp.left_shift(hi, _IDX_BITS), idxv)
        plsc.store_scatter(buf1, [pos], packed)
        plsc.store_scatter(hl, [lo], pos + 1, mask=ml)

    def pass1_body(i, c):
        va = nbuf[pl.ds(i * _LANES, _LANES)]
        vb = nbuf[pl.ds(t2 + i * _LANES, _LANES)]
        ika = (va * _SCALE).astype(jnp.int32)
        ikb = (vb * _SCALE).astype(jnp.int32)
        pass1_half(va, jnp.bitwise_and(ika, _NLO - 1),
                   jnp.right_shift(ika, _LO_BITS), lanes + i * _LANES, hla)
        pass1_half(vb, jnp.bitwise_and(ikb, _NLO - 1),
                   jnp.right_shift(ikb, _LO_BITS), lanes + t2 + i * _LANES,
                   hlb)
        return c

    lax.fori_loop(0, nv2, pass1_body, 0, unroll=8)

    @plsc.parallel_loop(0, nv2, unroll=4)
    def hist_hi_body(i):
        pa = buf1[pl.ds(i * _LANES, _LANES)]
        pb = buf1[pl.ds(t2 + i * _LANES, _LANES)]
        hia = jnp.right_shift(pa, _IDX_BITS)
        hib = jnp.right_shift(pb, _IDX_BITS)
        ca, ma = plsc.scan_count(hia)
        plsc.addupdate_scatter(hha, [hia], ca, mask=ma)
        cb, mb = plsc.scan_count(hib)
        plsc.addupdate_scatter(hhb, [hib], cb, mask=mb)

    lax.fori_loop(0, _NHI // _LANES, functools.partial(scan_body, hha, hhb),
                  jnp.int32(0))

    def pass2_half(p, hh):
        hi = jnp.right_shift(p, _IDX_BITS)
        idxv = jnp.bitwise_and(p, (1 << _IDX_BITS) - 1)
        cnt, mh = plsc.scan_count(hi)
        base = plsc.load_gather(hh, [hi])
        rank = base + cnt - 1
        plsc.store_scatter(rankb, [idxv], rank)
        plsc.store_scatter(keepb, [rank], idxv, mask=rank < n_keep)
        plsc.store_scatter(nbuf, [idxv],
                           jnp.where(rank >= n_keep, ones_f, zeros_f))
        plsc.store_scatter(hh, [hi], rank + 1, mask=mh)

    def pass2_body(i, c):
        pass2_half(buf1[pl.ds(i * _LANES, _LANES)], hha)
        pass2_half(buf1[pl.ds(t2 + i * _LANES, _LANES)], hhb)
        return c

    lax.fori_loop(0, nv2, pass2_body, 0, unroll=8)


def _make_sc_kernel(b, t):
    n_keep = t // 2
    rows_per_w = b // 32
    mesh = plsc.VectorSubcoreMesh(core_axis_name="c", subcore_axis_name="s")

    @functools.partial(
        pl.kernel,
        out_type=(
            jax.ShapeDtypeStruct((b, n_keep), jnp.int32),
            jax.ShapeDtypeStruct((b, t), jnp.int32),
            jax.ShapeDtypeStruct((b, t), jnp.float32),
        ),
        mesh=mesh,
        scratch_types=[
            pltpu.VMEM((t,), jnp.float32),
            pltpu.VMEM((t,), jnp.int32),
            pltpu.VMEM((t,), jnp.int32),
            pltpu.VMEM((n_keep,), jnp.int32),
            pltpu.VMEM((_NLO,), jnp.int32),
            pltpu.VMEM((_NLO,), jnp.int32),
            pltpu.VMEM((_NHI,), jnp.int32),
            pltpu.VMEM((_NHI,), jnp.int32),
        ],
        compiler_params=pltpu.CompilerParams(needs_layout_passes=False),
    )
    def sc_kernel(noise_hbm, keep_o, restore_o, mask_o,
                  nbuf, buf1, rankb, keepb, hla, hlb, hha, hhb):
        wid = lax.axis_index("s") * 2 + lax.axis_index("c")

        def do_row(r, c):
            row = wid * rows_per_w + r
            pltpu.sync_copy(noise_hbm.at[row], nbuf)
            _row_body(nbuf, buf1, rankb, keepb, hla, hlb, hha, hhb, t,
                      n_keep)
            pltpu.sync_copy(rankb, restore_o.at[row])
            pltpu.sync_copy(keepb, keep_o.at[row])
            pltpu.sync_copy(nbuf, mask_o.at[row])
            return c

        lax.fori_loop(0, rows_per_w, do_row, 0)

    return sc_kernel


def kernel(B, T, noise):
    b, t = noise.shape
    return _make_sc_kernel(b, t)(noise)
